# baseline (speedup 1.0000x reference)
import jax
import jax.numpy as jnp
from jax.experimental import pallas as pl
from jax.experimental.pallas import tpu as pltpu

_EPS = 1e-06
_ALPHA = 0.1
_D = 4


def _stats_half(s, s2, oh, n_sp):
    mu = s / n_sp
    var = (s2 - n_sp * mu * mu) / (n_sp - 1.0)
    std = jnp.sqrt(var + _EPS)
    c_sum = jax.lax.dot_general(s, oh, (((1,), (0,)), ((), ())),
                                preferred_element_type=jnp.float32)
    c_sum2 = jax.lax.dot_general(s2, oh, (((1,), (0,)), ((), ())),
                                 preferred_element_type=jnp.float32)
    return mu, std, c_sum, c_sum2


def _fused_body(cm_ref, lm_ref, x_ref, o_ref,
                in_buf, out_buf, s_t, s2_t, sc_t, bi_t, in_sem, out_sem):
    D, C2, HW = in_buf.shape
    C = C2 // 2
    P = lm_ref.shape[1]
    CH = x_ref.shape[0]

    def in_copy(b, j):
        return pltpu.make_async_copy(
            x_ref.at[pl.ds(b, 1)], in_buf.at[pl.ds(j, 1)], in_sem.at[j])

    def out_copy(b, j):
        return pltpu.make_async_copy(
            out_buf.at[pl.ds(j, 1)], o_ref.at[pl.ds(b, 1)], out_sem.at[j])

    lane = jax.lax.broadcasted_iota(jnp.int32, (1, P), 1)

    for j in range(D):
        in_copy(j, j).start()

    s_t[...] = jnp.zeros_like(s_t)
    s2_t[...] = jnp.zeros_like(s2_t)

    def step_a(i, carry):
        j = jax.lax.rem(i, D)
        in_copy(i, j).wait()
        xc = in_buf[pl.ds(j, 1)][0]
        mask = (lane == i).astype(jnp.float32)
        s_t[...] += jnp.sum(xc, axis=1, keepdims=True) * mask
        s2_t[...] += jnp.sum(xc * xc, axis=1, keepdims=True) * mask

        @pl.when(i + D < CH)
        def _():
            in_copy(i + D, j).start()
        return carry

    jax.lax.fori_loop(0, CH, step_a, 0)

    for j in range(D):
        in_copy(j, j).start()

    cm_e = cm_ref[0]
    cm_o = cm_ref[1]
    lm_e = lm_ref[pl.ds(0, 1)]
    lm_o = lm_ref[pl.ds(1, 1)]
    K = cm_e.shape[1]

    kiota = jax.lax.broadcasted_iota(jnp.int32, (P, K), 1)
    ids_e = jnp.argmax(cm_e, axis=1)
    ids_o = jnp.argmax(cm_o, axis=1)
    oh_e = (ids_e[:, None] == kiota).astype(jnp.float32)
    oh_o = (ids_o[:, None] == kiota).astype(jnp.float32)

    n_sp = jnp.float32(HW)
    s_e, s_o = s_t[pl.ds(0, 128)], s_t[pl.ds(128, 128)]
    s2_e, s2_o = s2_t[pl.ds(0, 128)], s2_t[pl.ds(128, 128)]

    mu_e, std_e, csum_e, csum2_e = _stats_half(s_e, s2_e, oh_e, n_sp)
    mu_o, std_o, csum_o, csum2_o = _stats_half(s_o, s2_o, oh_o, n_sp)

    counts = jnp.sum(oh_e, axis=0) + jnp.sum(oh_o, axis=0)
    c_sum = csum_e + csum_o
    c_sum2 = csum2_e + csum2_o
    n_c = counts * n_sp
    n_c_safe = jnp.maximum(n_c, 1.0)[None, :]
    denom = jnp.maximum(n_c - 1.0, 1.0)[None, :]
    cmu_k = c_sum / n_c_safe
    cvar_k = (c_sum2 - n_c[None, :] * cmu_k * cmu_k) / denom
    cstd_k = jnp.sqrt(jnp.maximum(cvar_k, 0.0) + _EPS)

    def back(tab, oh):
        return jax.lax.dot_general(tab, oh, (((1,), (1,)), ((), ())),
                                   preferred_element_type=jnp.float32)

    def mixed(mu, std, oh, lm):
        cmu = back(cmu_k, oh)
        cstd = back(cstd_k, oh)
        mu_mix = mu * lm + cmu * (1.0 - lm)
        std_mix = std * lm + cstd * (1.0 - lm)
        scale = std_mix / std
        return scale, mu_mix - mu * scale

    scale_e, bias_e = mixed(mu_e, std_e, oh_e, lm_e)
    scale_o, bias_o = mixed(mu_o, std_o, oh_o, lm_o)
    sc_t[pl.ds(0, 128)] = scale_e
    sc_t[pl.ds(128, 128)] = scale_o
    bi_t[pl.ds(0, 128)] = bias_e
    bi_t[pl.ds(128, 128)] = bias_o

    def step_c(i, carry):
        j = jax.lax.rem(i, D)
        in_copy(i, j).wait()

        @pl.when(i >= D)
        def _():
            out_copy(i - D, j).wait()

        xc = in_buf[pl.ds(j, 1)]
        mask = (lane == i).astype(jnp.float32)
        sc = jnp.sum(sc_t[...] * mask, axis=1, keepdims=True)
        bi = jnp.sum(bi_t[...] * mask, axis=1, keepdims=True)
        out_buf[pl.ds(j, 1)] = xc * sc + bi
        out_copy(i, j).start()

        @pl.when(i + D < CH)
        def _():
            in_copy(i + D, j).start()
        return carry

    jax.lax.fori_loop(0, CH, step_c, 0)

    for i in range(CH - D, CH):
        out_copy(i, i % D).wait()


def kernel(x, cluster_map):
    B, C, H, W = x.shape
    HW = H * W
    P = B // 2
    xf = x.reshape(P, 2 * C, HW)

    lmda = jax.random.beta(jax.random.key(42), _ALPHA, _ALPHA, (B, 1, 1, 1)).astype(x.dtype)
    lm = lmda.reshape(P, 2).T

    cm = cluster_map[0]
    cm2 = cm.reshape(P, 2, cm.shape[1]).transpose(1, 0, 2)

    out = pl.pallas_call(
        _fused_body,
        in_specs=[
            pl.BlockSpec(memory_space=pltpu.MemorySpace.VMEM),
            pl.BlockSpec(memory_space=pltpu.MemorySpace.VMEM),
            pl.BlockSpec(memory_space=pltpu.MemorySpace.HBM),
        ],
        out_specs=pl.BlockSpec(memory_space=pltpu.MemorySpace.HBM),
        out_shape=jax.ShapeDtypeStruct((P, 2 * C, HW), x.dtype),
        scratch_shapes=[
            pltpu.VMEM((_D, 2 * C, HW), jnp.float32),
            pltpu.VMEM((_D, 2 * C, HW), jnp.float32),
            pltpu.VMEM((2 * C, P), jnp.float32),
            pltpu.VMEM((2 * C, P), jnp.float32),
            pltpu.VMEM((2 * C, P), jnp.float32),
            pltpu.VMEM((2 * C, P), jnp.float32),
            pltpu.SemaphoreType.DMA((_D,)),
            pltpu.SemaphoreType.DMA((_D,)),
        ],
    )(cm2, lm, xf)

    return out.reshape(B, C, H, W)


# speedup vs baseline: 1.7702x; 1.7702x over previous
import jax
import jax.numpy as jnp
from jax.experimental import pallas as pl
from jax.experimental.pallas import tpu as pltpu

_EPS = 1e-06
_ALPHA = 0.1
_D = 12


def _fused_body(cm_ref, lm_ref, x_ref, o_ref,
                in_buf, out_buf, s_t, s2_t, sc_t, bi_t, in_sem, out_sem):
    D, C, HW = in_buf.shape
    B = lm_ref.shape[1]
    CH = x_ref.shape[0]

    def in_copy(b, j):
        return pltpu.make_async_copy(
            x_ref.at[pl.ds(b, 1)], in_buf.at[pl.ds(j, 1)], in_sem.at[j])

    def out_copy(b, j):
        return pltpu.make_async_copy(
            out_buf.at[pl.ds(j, 1)], o_ref.at[pl.ds(b, 1)], out_sem.at[j])

    lane = jax.lax.broadcasted_iota(jnp.int32, (1, B), 1)

    for j in range(D):
        in_copy(j, j).start()

    s_t[...] = jnp.zeros_like(s_t)
    s2_t[...] = jnp.zeros_like(s2_t)

    def step_a(i, carry):
        j = jax.lax.rem(i, D)
        in_copy(i, j).wait()
        xc = in_buf[pl.ds(j, 1)][0]
        mask = (lane == i).astype(jnp.float32)
        s_t[...] += jnp.sum(xc, axis=1, keepdims=True) * mask
        s2_t[...] += jnp.sum(xc * xc, axis=1, keepdims=True) * mask

        @pl.when(i + D < CH)
        def _():
            in_copy(i + D, j).start()
        return carry

    jax.lax.fori_loop(0, CH, step_a, 0)

    for j in range(D):
        in_copy(j, j).start()

    cm = cm_ref[0]
    lm = lm_ref[...]
    K = cm.shape[1]
    s = s_t[...]
    s2 = s2_t[...]

    ids = jnp.argmax(cm, axis=1)
    onehot = (ids[:, None] == jax.lax.broadcasted_iota(jnp.int32, (B, K), 1)
              ).astype(jnp.float32)

    n_sp = jnp.float32(HW)
    mu = s / n_sp
    var = (s2 - n_sp * mu * mu) / (n_sp - 1.0)
    std = jnp.sqrt(var + _EPS)

    counts = jnp.sum(onehot, axis=0)
    c_sum = jax.lax.dot_general(s, onehot, (((1,), (0,)), ((), ())),
                                preferred_element_type=jnp.float32)
    c_sum2 = jax.lax.dot_general(s2, onehot, (((1,), (0,)), ((), ())),
                                 preferred_element_type=jnp.float32)
    n_c = counts * n_sp
    n_c_safe = jnp.maximum(n_c, 1.0)[None, :]
    denom = jnp.maximum(n_c - 1.0, 1.0)[None, :]
    cmu_k = c_sum / n_c_safe
    cvar_k = (c_sum2 - n_c[None, :] * cmu_k * cmu_k) / denom
    cstd_k = jnp.sqrt(jnp.maximum(cvar_k, 0.0) + _EPS)

    cmu = jax.lax.dot_general(cmu_k, onehot, (((1,), (1,)), ((), ())),
                              preferred_element_type=jnp.float32)
    cstd = jax.lax.dot_general(cstd_k, onehot, (((1,), (1,)), ((), ())),
                               preferred_element_type=jnp.float32)

    mu_mix = mu * lm + cmu * (1.0 - lm)
    std_mix = std * lm + cstd * (1.0 - lm)
    scale = std_mix / std
    sc_t[...] = scale
    bi_t[...] = mu_mix - mu * scale

    def step_c(i, carry):
        j = jax.lax.rem(i, D)
        in_copy(i, j).wait()

        @pl.when(i >= D)
        def _():
            out_copy(i - D, j).wait()

        xc = in_buf[pl.ds(j, 1)]
        mask = (lane == i).astype(jnp.float32)
        sc = jnp.sum(sc_t[...] * mask, axis=1, keepdims=True)
        bi = jnp.sum(bi_t[...] * mask, axis=1, keepdims=True)
        out_buf[pl.ds(j, 1)] = xc * sc + bi
        out_copy(i, j).start()

        @pl.when(i + D < CH)
        def _():
            in_copy(i + D, j).start()
        return carry

    jax.lax.fori_loop(0, CH, step_c, 0)

    for i in range(CH - D, CH):
        out_copy(i, i % D).wait()


def kernel(x, cluster_map):
    B, C, H, W = x.shape
    HW = H * W
    xf = x.reshape(B, C, HW)

    lmda = jax.random.beta(jax.random.key(42), _ALPHA, _ALPHA, (B, 1, 1, 1)).astype(x.dtype)
    lm = lmda.reshape(1, B)

    out = pl.pallas_call(
        _fused_body,
        in_specs=[
            pl.BlockSpec(memory_space=pltpu.MemorySpace.VMEM),
            pl.BlockSpec(memory_space=pltpu.MemorySpace.VMEM),
            pl.BlockSpec(memory_space=pltpu.MemorySpace.HBM),
        ],
        out_specs=pl.BlockSpec(memory_space=pltpu.MemorySpace.HBM),
        out_shape=jax.ShapeDtypeStruct((B, C, HW), x.dtype),
        scratch_shapes=[
            pltpu.VMEM((_D, C, HW), jnp.float32),
            pltpu.VMEM((_D, C, HW), jnp.float32),
            pltpu.VMEM((C, B), jnp.float32),
            pltpu.VMEM((C, B), jnp.float32),
            pltpu.VMEM((C, B), jnp.float32),
            pltpu.VMEM((C, B), jnp.float32),
            pltpu.SemaphoreType.DMA((_D,)),
            pltpu.SemaphoreType.DMA((_D,)),
        ],
    )(cluster_map, lm, xf)

    return out.reshape(B, C, H, W)


# speedup vs baseline: 2.0837x; 1.1771x over previous
import jax
import jax.numpy as jnp
from jax.experimental import pallas as pl
from jax.experimental.pallas import tpu as pltpu

_EPS = 1e-06
_ALPHA = 0.1
_D = 8


def _fused_body(cm_ref, lm_ref, x_ref, o_ref,
                in_buf, out_buf, s_t, s2_t, sc_t, bi_t, in_sem, out_sem):
    D, C, HW = in_buf.shape
    B = lm_ref.shape[1]
    CH = x_ref.shape[0]

    def in_copy(b, j):
        return pltpu.make_async_copy(
            x_ref.at[pl.ds(b, 1)], in_buf.at[pl.ds(j, 1)], in_sem.at[j])

    def out_copy(b, j):
        return pltpu.make_async_copy(
            out_buf.at[pl.ds(j, 1)], o_ref.at[pl.ds(b, 1)], out_sem.at[j])

    lane = jax.lax.broadcasted_iota(jnp.int32, (1, B), 1)

    for j in range(D):
        in_copy(j, j).start()

    s_t[...] = jnp.zeros_like(s_t)
    s2_t[...] = jnp.zeros_like(s2_t)

    def step_a(i, carry):
        j = jax.lax.rem(i, D)
        in_copy(i, j).wait()
        xc = in_buf[pl.ds(j, 1)][0]
        mask = (lane == i).astype(jnp.float32)
        s_t[...] += jnp.sum(xc, axis=1, keepdims=True) * mask
        s2_t[...] += jnp.sum(xc * xc, axis=1, keepdims=True) * mask

        @pl.when(i + D < CH)
        def _():
            in_copy(i + D, j).start()
        return carry

    jax.lax.fori_loop(0, CH, step_a, 0)

    for j in range(D):
        in_copy(j, j).start()

    cm = cm_ref[0]
    lm = lm_ref[...]
    K = cm.shape[1]
    s = s_t[...]
    s2 = s2_t[...]

    ids = jnp.argmax(cm, axis=1)
    onehot = (ids[:, None] == jax.lax.broadcasted_iota(jnp.int32, (B, K), 1)
              ).astype(jnp.float32)

    n_sp = jnp.float32(HW)
    mu = s / n_sp
    var = (s2 - n_sp * mu * mu) / (n_sp - 1.0)
    std = jnp.sqrt(var + _EPS)

    counts = jnp.sum(onehot, axis=0)
    c_sum = jax.lax.dot_general(s, onehot, (((1,), (0,)), ((), ())),
                                preferred_element_type=jnp.float32)
    c_sum2 = jax.lax.dot_general(s2, onehot, (((1,), (0,)), ((), ())),
                                 preferred_element_type=jnp.float32)
    n_c = counts * n_sp
    n_c_safe = jnp.maximum(n_c, 1.0)[None, :]
    denom = jnp.maximum(n_c - 1.0, 1.0)[None, :]
    cmu_k = c_sum / n_c_safe
    cvar_k = (c_sum2 - n_c[None, :] * cmu_k * cmu_k) / denom
    cstd_k = jnp.sqrt(jnp.maximum(cvar_k, 0.0) + _EPS)

    cmu = jax.lax.dot_general(cmu_k, onehot, (((1,), (1,)), ((), ())),
                              preferred_element_type=jnp.float32)
    cstd = jax.lax.dot_general(cstd_k, onehot, (((1,), (1,)), ((), ())),
                               preferred_element_type=jnp.float32)

    mu_mix = mu * lm + cmu * (1.0 - lm)
    std_mix = std * lm + cstd * (1.0 - lm)
    scale = std_mix / std
    sc_t[...] = scale
    bi_t[...] = mu_mix - mu * scale

    def step_c(i, carry):
        j = jax.lax.rem(i, D)
        in_copy(i, j).wait()

        @pl.when(i >= D)
        def _():
            out_copy(i - D, j).wait()

        xc = in_buf[pl.ds(j, 1)]
        mask = (lane == i).astype(jnp.float32)
        sc = jnp.sum(sc_t[...] * mask, axis=1, keepdims=True)
        bi = jnp.sum(bi_t[...] * mask, axis=1, keepdims=True)
        out_buf[pl.ds(j, 1)] = xc * sc + bi
        out_copy(i, j).start()

        @pl.when(i + D < CH)
        def _():
            in_copy(i + D, j).start()
        return carry

    jax.lax.fori_loop(0, CH, step_c, 0)

    for i in range(CH - D, CH):
        out_copy(i, i % D).wait()


def kernel(x, cluster_map):
    B, C, H, W = x.shape
    HW = H * W
    xf = x.reshape(B, C, HW)

    with jax.ensure_compile_time_eval():
        lmda = jax.random.beta(
            jax.random.key(42), _ALPHA, _ALPHA, (B, 1, 1, 1)).astype(x.dtype)
    lm = lmda.reshape(1, B)

    out = pl.pallas_call(
        _fused_body,
        in_specs=[
            pl.BlockSpec(memory_space=pltpu.MemorySpace.VMEM),
            pl.BlockSpec(memory_space=pltpu.MemorySpace.VMEM),
            pl.BlockSpec(memory_space=pltpu.MemorySpace.HBM),
        ],
        out_specs=pl.BlockSpec(memory_space=pltpu.MemorySpace.HBM),
        out_shape=jax.ShapeDtypeStruct((B, C, HW), x.dtype),
        scratch_shapes=[
            pltpu.VMEM((_D, C, HW), jnp.float32),
            pltpu.VMEM((_D, C, HW), jnp.float32),
            pltpu.VMEM((C, B), jnp.float32),
            pltpu.VMEM((C, B), jnp.float32),
            pltpu.VMEM((C, B), jnp.float32),
            pltpu.VMEM((C, B), jnp.float32),
            pltpu.SemaphoreType.DMA((_D,)),
            pltpu.SemaphoreType.DMA((_D,)),
        ],
    )(cluster_map, lm, xf)

    return out.reshape(B, C, H, W)
